# baseline (speedup 1.0000x reference)
import functools

import jax
import jax.numpy as jnp
from jax import lax
from jax.experimental import pallas as pl
from jax.experimental.pallas import tpu as pltpu
from jax.experimental.pallas import tpu_sc as plsc

_K = 32
_N0 = 4096
_N1 = 200
_B = _N0 * _N1
_NW = 32
_PERW = _B // _NW
_CHUNK = 40
_ROWCH = _N1 // _CHUNK
_GOUT = 4
_GROUPS = (_N0 // _NW) // _GOUT
_PAIRS = _GROUPS // 2
_WCH = _PERW // _CHUNK


def _prologue_body(mask_ref, logits_ref, idx_ref, table_ref, adj_ref):
    masked = jnp.where(mask_ref[...] == 0.0, jnp.float32(-50.0), logits_ref[...])
    m = jnp.max(masked, axis=1, keepdims=True)
    s = masked - m
    lp = s - jnp.log(jnp.sum(jnp.exp(s), axis=1, keepdims=True))
    table_ref[...] = jnp.broadcast_to(lp[None], (_NW, _K, _K))
    row = lax.broadcasted_iota(jnp.int32, idx_ref.shape, 0)
    adj_ref[...] = idx_ref[...] + (row // _WCH) * _K


def _prologue(mask, logits, idx2d):
    return pl.pallas_call(
        _prologue_body,
        out_shape=(
            jax.ShapeDtypeStruct((_NW, _K, _K), jnp.float32),
            jax.ShapeDtypeStruct((_NW * _WCH, _CHUNK), jnp.int32),
        ),
    )(mask, logits, idx2d)


def _sc_gather(table_rep, idx2d):
    mesh = plsc.VectorSubcoreMesh(core_axis_name="c", subcore_axis_name="s")

    @functools.partial(
        pl.kernel,
        mesh=mesh,
        out_type=jax.ShapeDtypeStruct((_N0, _N1, _K), jnp.float32),
        scratch_types=[
            pltpu.VMEM((_WCH, _CHUNK), jnp.int32),
            pltpu.VMEM((2, _GOUT, _N1, _K), jnp.float32),
            pltpu.SemaphoreType.DMA,
            pltpu.SemaphoreType.DMA,
            pltpu.SemaphoreType.DMA,
            pltpu.SemaphoreType.DMA,
        ],
        compiler_params=pltpu.CompilerParams(
            use_tc_tiling_on_sc=False, needs_layout_passes=False
        ),
    )
    def k(table_hbm, idx_hbm, out_hbm, idx_v, rows_v, g0, g1, o0, o1):
        gsem = (g0, g1)
        osem = (o0, o1)
        wid = lax.axis_index("s") * 2 + lax.axis_index("c")
        out0 = wid * (_N0 // _NW)
        pltpu.sync_copy(idx_hbm.at[pl.ds(wid * _WCH, _WCH)], idx_v)

        def out_slice(g):
            return out_hbm.at[pl.ds(out0 + g * _GOUT, _GOUT)]

        def fire(g, b):
            for r in range(_GOUT):
                for c in range(_ROWCH):
                    j = (g * _GOUT + r) * _ROWCH + c
                    pltpu.async_copy(
                        table_hbm.at[idx_v.at[j]],
                        rows_v.at[b, r, pl.ds(c * _CHUNK, _CHUNK)],
                        gsem[b],
                    )

        def wait_gathers(g, b):
            pltpu.make_async_copy(out_slice(g), rows_v.at[b], gsem[b]).wait()

        def start_out(g, b):
            pltpu.async_copy(rows_v.at[b], out_slice(g), osem[b])

        def wait_out(g, b):
            pltpu.make_async_copy(rows_v.at[b], out_slice(g), osem[b]).wait()

        def pair(p, carry):
            for b in (0, 1):
                g = 2 * p + b

                @pl.when(p > 0)
                def _():
                    wait_out(g, b)

                fire(g, b)
            for b in (0, 1):
                g = 2 * p + b
                wait_gathers(g, b)
                start_out(g, b)
            return carry

        lax.fori_loop(0, _PAIRS, pair, 0)
        last = 2 * _PAIRS - 2
        wait_out(last, 0)
        wait_out(last + 1, 1)

    return k(table_rep, idx2d)


def kernel(prev_labels, mask, logits):
    idx2d = prev_labels.astype(jnp.int32).reshape(_NW * _WCH, _CHUNK)
    table_rep, idx_adj = _prologue(
        mask.astype(jnp.float32), logits.astype(jnp.float32), idx2d
    )
    return _sc_gather(table_rep.reshape(_NW * _K, _K), idx_adj)


# speedup vs baseline: 1.6905x; 1.6905x over previous
import functools

import jax
import jax.numpy as jnp
from jax import lax
from jax.experimental import pallas as pl
from jax.experimental.pallas import tpu as pltpu
from jax.experimental.pallas import tpu_sc as plsc

_K = 32
_N0 = 4096
_N1 = 200
_NW = 32
_LANES = _N0 // _NW
_PAIRS = _N1 // 2


def _prologue_body(mask_ref, logits_ref, idx_ref, tab_ref, idxt_ref):
    masked = jnp.where(mask_ref[...] == 0.0, jnp.float32(-50.0), logits_ref[...])
    m = jnp.max(masked, axis=1, keepdims=True)
    s = masked - m
    lp = s - jnp.log(jnp.sum(jnp.exp(s), axis=1, keepdims=True))
    tab_ref[...] = lp.T
    idxt_ref[...] = jnp.transpose(
        idx_ref[...].reshape(_NW, _LANES, _N1), (0, 2, 1)
    )


def _prologue(mask, logits, idx):
    return pl.pallas_call(
        _prologue_body,
        out_shape=(
            jax.ShapeDtypeStruct((_K, _K), jnp.float32),
            jax.ShapeDtypeStruct((_NW, _N1, _LANES), jnp.int32),
        ),
    )(mask, logits, idx)


def _sc_expand(tab_flat, idxt):
    mesh = plsc.VectorSubcoreMesh(core_axis_name="c", subcore_axis_name="s")

    @functools.partial(
        pl.kernel,
        mesh=mesh,
        out_type=jax.ShapeDtypeStruct((_N1, _K, _N0), jnp.float32),
        scratch_types=[
            pltpu.VMEM((_K * _K,), jnp.float32),
            pltpu.VMEM((_N1 * _LANES,), jnp.int32),
            pltpu.VMEM((2, _K, _LANES), jnp.float32),
            pltpu.SemaphoreType.DMA,
            pltpu.SemaphoreType.DMA,
        ],
        compiler_params=pltpu.CompilerParams(
            use_tc_tiling_on_sc=False, needs_layout_passes=False
        ),
    )
    def k(tab_hbm, idxt_hbm, out_hbm, tab_v, idx_v, buf_v, o0, o1):
        osem = (o0, o1)
        wid = lax.axis_index("s") * 2 + lax.axis_index("c")
        lane0 = wid * _LANES
        pltpu.sync_copy(idxt_hbm.at[wid], idx_v)
        pltpu.sync_copy(tab_hbm, tab_v)

        def out_slice(j):
            return out_hbm.at[j, :, pl.ds(lane0, _LANES)]

        def build(j, b):
            for lg in range(_LANES // 16):
                idxv = idx_v[pl.ds(j * _LANES + lg * 16, 16)]
                for c in range(_K):
                    vals = plsc.load_gather(tab_v, [idxv + c * _K])
                    buf_v[b, c, pl.ds(lg * 16, 16)] = vals

        def start_out(j, b):
            pltpu.async_copy(buf_v.at[b], out_slice(j), osem[b])

        def wait_out(j, b):
            pltpu.make_async_copy(buf_v.at[b], out_slice(j), osem[b]).wait()

        def pair(p, carry):
            for b in (0, 1):
                j = 2 * p + b

                @pl.when(p > 0)
                def _():
                    wait_out(j, b)

                build(j, b)
                start_out(j, b)
            return carry

        lax.fori_loop(0, _PAIRS, pair, 0)
        last = 2 * _PAIRS - 2
        wait_out(last, 0)
        wait_out(last + 1, 1)

    return k(tab_flat, idxt)


def kernel(prev_labels, mask, logits):
    tab, idxt = _prologue(
        mask.astype(jnp.float32), logits.astype(jnp.float32),
        prev_labels.astype(jnp.int32),
    )
    out_t = _sc_expand(tab.reshape(_K * _K), idxt.reshape(_NW, _N1 * _LANES))
    return out_t.transpose(2, 0, 1)


# speedup vs baseline: 2.2436x; 1.3272x over previous
import functools

import jax
import jax.numpy as jnp
from jax import lax
from jax.experimental import pallas as pl
from jax.experimental.pallas import tpu as pltpu
from jax.experimental.pallas import tpu_sc as plsc

_K = 32
_N0 = 4096
_N1 = 200
_NW = 32
_LANES = _N0 // _NW
_PAIRS = _N1 // 2


def _prologue_body(mask_ref, logits_ref, idx_ref, tab_ref, idxt_ref):
    masked = jnp.where(mask_ref[...] == 0.0, jnp.float32(-50.0), logits_ref[...])
    m = jnp.max(masked, axis=1, keepdims=True)
    s = masked - m
    lp = s - jnp.log(jnp.sum(jnp.exp(s), axis=1, keepdims=True))
    tab_ref[...] = lp.T
    idxt_ref[...] = jnp.transpose(
        idx_ref[...].reshape(_NW, _LANES, _N1), (0, 2, 1)
    )


def _prologue(mask, logits, idx):
    return pl.pallas_call(
        _prologue_body,
        out_shape=(
            jax.ShapeDtypeStruct((_K, _K), jnp.float32),
            jax.ShapeDtypeStruct((_NW, _N1, _LANES), jnp.int32),
        ),
    )(mask, logits, idx)


def _sc_expand(tab_flat, idxt):
    mesh = plsc.VectorSubcoreMesh(core_axis_name="c", subcore_axis_name="s")

    @functools.partial(
        pl.kernel,
        mesh=mesh,
        out_type=jax.ShapeDtypeStruct((_N1, _K, _N0), jnp.float32),
        scratch_types=[
            pltpu.VMEM((_K * _K,), jnp.float32),
            pltpu.VMEM((_N1 * _LANES,), jnp.int32),
            pltpu.VMEM((2, _K, _LANES), jnp.float32),
            pltpu.SemaphoreType.DMA,
            pltpu.SemaphoreType.DMA,
        ],
        compiler_params=pltpu.CompilerParams(
            use_tc_tiling_on_sc=True, needs_layout_passes=False
        ),
    )
    def k(tab_hbm, idxt_hbm, out_hbm, tab_v, idx_v, buf_v, o0, o1):
        osem = (o0, o1)
        wid = lax.axis_index("s") * 2 + lax.axis_index("c")
        lane0 = wid * _LANES
        pltpu.sync_copy(idxt_hbm.at[wid], idx_v)
        pltpu.sync_copy(tab_hbm, tab_v)

        def out_slice(j):
            return out_hbm.at[j, :, pl.ds(lane0, _LANES)]

        def build(j, b):
            for lg in range(_LANES // 16):
                idxv = idx_v[pl.ds(j * _LANES + lg * 16, 16)]
                for c in range(_K):
                    vals = plsc.load_gather(tab_v, [idxv + c * _K])
                    buf_v[b, c, pl.ds(lg * 16, 16)] = vals

        def start_out(j, b):
            pltpu.async_copy(buf_v.at[b], out_slice(j), osem[b])

        def wait_out(j, b):
            pltpu.make_async_copy(buf_v.at[b], out_slice(j), osem[b]).wait()

        def pair(p, carry):
            for b in (0, 1):
                j = 2 * p + b

                @pl.when(p > 0)
                def _():
                    wait_out(j, b)

                build(j, b)
                start_out(j, b)
            return carry

        lax.fori_loop(0, _PAIRS, pair, 0)
        last = 2 * _PAIRS - 2
        wait_out(last, 0)
        wait_out(last + 1, 1)

    return k(tab_flat, idxt)


def kernel(prev_labels, mask, logits):
    tab, idxt = _prologue(
        mask.astype(jnp.float32), logits.astype(jnp.float32),
        prev_labels.astype(jnp.int32),
    )
    out_t = _sc_expand(tab.reshape(_K * _K), idxt.reshape(_NW, _N1 * _LANES))
    return out_t.transpose(2, 0, 1)


# speedup vs baseline: 6.4037x; 2.8542x over previous
import functools

import jax
import jax.numpy as jnp
from jax import lax
from jax.experimental import pallas as pl
from jax.experimental.pallas import tpu as pltpu
from jax.experimental.pallas import tpu_sc as plsc

_K = 32
_N0 = 4096
_N1 = 200
_NW = 32
_LANES = _N0 // _NW
_PAIRS = _N1 // 2


def _prologue_body(mask_ref, logits_ref, idx_ref, tab_ref, idxt_ref):
    masked = jnp.where(mask_ref[...] == 0.0, jnp.float32(-50.0), logits_ref[...])
    m = jnp.max(masked, axis=1, keepdims=True)
    s = masked - m
    lp = s - jnp.log(jnp.sum(jnp.exp(s), axis=1, keepdims=True))
    tab_ref[...] = lp.T
    idxt_ref[...] = jnp.transpose(
        idx_ref[...].reshape(_NW, _LANES, _N1), (0, 2, 1)
    )


def _prologue(mask, logits, idx):
    return pl.pallas_call(
        _prologue_body,
        out_shape=(
            jax.ShapeDtypeStruct((_K, _K), jnp.float32),
            jax.ShapeDtypeStruct((_NW, _N1, _LANES), jnp.int32),
        ),
    )(mask, logits, idx)


def _sc_expand(tab_flat, idxt):
    mesh = plsc.VectorSubcoreMesh(core_axis_name="c", subcore_axis_name="s")

    @functools.partial(
        pl.kernel,
        mesh=mesh,
        out_type=jax.ShapeDtypeStruct((_N1, _K, _N0), jnp.float32),
        scratch_types=[
            pltpu.VMEM((_K * _K,), jnp.float32),
            pltpu.VMEM((_N1 * _LANES,), jnp.int32),
            pltpu.VMEM((2, _K, _LANES), jnp.float32),
            pltpu.SemaphoreType.DMA,
            pltpu.SemaphoreType.DMA,
        ],
        compiler_params=pltpu.CompilerParams(
            use_tc_tiling_on_sc=True, needs_layout_passes=False
        ),
    )
    def k(tab_hbm, idxt_hbm, out_hbm, tab_v, idx_v, buf_v, o0, o1):
        osem = (o0, o1)
        wid = lax.axis_index("s") * 2 + lax.axis_index("c")
        lane0 = wid * _LANES
        pltpu.sync_copy(idxt_hbm.at[wid], idx_v)
        pltpu.sync_copy(tab_hbm, tab_v)

        def out_slice(j):
            return out_hbm.at[j, :, pl.ds(lane0, _LANES)]

        def build(j, b):
            for lg in range(_LANES // 16):
                idxv = idx_v[pl.ds(j * _LANES + lg * 16, 16)]
                vals = [
                    plsc.load_gather(tab_v, [idxv + c * _K]) for c in range(_K)
                ]
                for c in range(_K):
                    buf_v[b, c, pl.ds(lg * 16, 16)] = vals[c]

        def start_out(j, b):
            pltpu.async_copy(buf_v.at[b], out_slice(j), osem[b])

        def wait_out(j, b):
            pltpu.make_async_copy(buf_v.at[b], out_slice(j), osem[b]).wait()

        def pair(p, carry):
            for b in (0, 1):
                j = 2 * p + b

                @pl.when(p > 0)
                def _():
                    wait_out(j, b)

                build(j, b)
                start_out(j, b)
            return carry

        lax.fori_loop(0, _PAIRS, pair, 0)
        last = 2 * _PAIRS - 2
        wait_out(last, 0)
        wait_out(last + 1, 1)

    return k(tab_flat, idxt)


def kernel(prev_labels, mask, logits):
    tab, idxt = _prologue(
        mask.astype(jnp.float32), logits.astype(jnp.float32),
        prev_labels.astype(jnp.int32),
    )
    out_t = _sc_expand(tab.reshape(_K * _K), idxt.reshape(_NW, _N1 * _LANES))
    return out_t.transpose(2, 0, 1)
